# baseline (speedup 1.0000x reference)
import jax
import jax.numpy as jnp
from jax import lax
from jax.experimental import pallas as pl
from jax.experimental.pallas import tpu as pltpu
from jax.experimental.pallas import tpu_sc as plsc

_N = 4_194_304
_G = 1000
_GPAD = 1024
_NC = 2
_NS = 16
_NW = _NC * _NS
_PER_TILE = _N // _NW
_CHUNK = 8192
_STEPS = _PER_TILE // _CHUNK
_L = 16


def _body(x_hbm, g_hbm, means_hbm, sds_hbm, out_hbm,
          tbl_m, tbl_s, tbl_p,
          x_v0, x_v1, g_v0, g_v1, o_v0, o_v1,
          semld0, semld1, semst0, semst1, semtb):
    wid = lax.axis_index("s") * _NC + lax.axis_index("c")
    base = wid * _PER_TILE
    x_v = (x_v0, x_v1)
    g_v = (g_v0, g_v1)
    o_v = (o_v0, o_v1)
    semld = (semld0, semld1)
    semst = (semst0, semst1)

    def load(c, b):
        off = base + c * _CHUNK
        cx = pltpu.make_async_copy(
            x_hbm.at[pl.ds(off, _CHUNK)], x_v[b], semld[b])
        cg = pltpu.make_async_copy(
            g_hbm.at[pl.ds(off, _CHUNK)], g_v[b], semld[b])
        return cx, cg

    def store(c, b):
        off = base + c * _CHUNK
        return pltpu.make_async_copy(
            o_v[b], out_hbm.at[pl.ds(off, _CHUNK)], semst[b])

    def start(copies):
        for cp in copies:
            cp.start()

    def wait(copies):
        for cp in copies:
            cp.wait()

    def compute(b):
        gr = g_v[b]
        xr = x_v[b]
        orf = o_v[b]

        @plsc.parallel_loop(0, _CHUNK, step=_L, unroll=4)
        def _inner(e):
            idx = gr[pl.ds(e, _L)] - 1
            p = plsc.load_gather(tbl_p, [idx])
            a = plsc.bitcast(p & jnp.int32(-65536), jnp.float32)
            bb = plsc.bitcast(lax.shift_left(p, jnp.int32(16)), jnp.float32)
            orf[pl.ds(e, _L)] = xr[pl.ds(e, _L)] * a + bb

    start(load(0, 0))
    start(load(1, 1))

    cm = pltpu.make_async_copy(means_hbm, tbl_m.at[pl.ds(0, _G)], semtb)
    cs = pltpu.make_async_copy(sds_hbm, tbl_s.at[pl.ds(0, _G)], semtb)
    start((cm, cs))
    wait((cm, cs))

    def init_tbl(i, carry):
        m = tbl_m[pl.ds(i * _L, _L)]
        s = tbl_s[pl.ds(i * _L, _L)]
        a = 1.0 / s
        b = -(m * a)
        ai = plsc.bitcast(a, jnp.int32) + jnp.int32(0x8000)
        bi = plsc.bitcast(b, jnp.int32) + jnp.int32(0x8000)
        packed = (ai & jnp.int32(-65536)) | (
            lax.shift_right_logical(bi, jnp.int32(16)))
        tbl_p[pl.ds(i * _L, _L)] = packed
        return carry

    lax.fori_loop(0, (_G + _L - 1) // _L, init_tbl, 0)


    wait(load(0, 0))
    compute(0)
    store(0, 0).start()

    start(load(2, 0))
    wait(load(1, 1))
    compute(1)
    store(1, 1).start()

    def steady(k, carry):
        c0 = 2 + 2 * k
        for b in range(2):
            c = c0 + b
            start(load(c + 1, 1 - b))
            wait(load(c, b))
            store(c - 2, b).wait()
            compute(b)
            store(c, b).start()
        return carry

    lax.fori_loop(0, (_STEPS - 4) // 2, steady, 0)

    start(load(_STEPS - 1, 1))
    wait(load(_STEPS - 2, 0))
    store(_STEPS - 4, 0).wait()
    compute(0)
    store(_STEPS - 2, 0).start()

    wait(load(_STEPS - 1, 1))
    store(_STEPS - 3, 1).wait()
    compute(1)
    store(_STEPS - 1, 1).start()

    store(_STEPS - 2, 0).wait()
    store(_STEPS - 1, 1).wait()


def kernel(x, group, means, sds):
    g32 = group.astype(jnp.int32)
    mesh = plsc.VectorSubcoreMesh(core_axis_name="c", subcore_axis_name="s")
    run = pl.kernel(
        _body,
        mesh=mesh,
        compiler_params=pltpu.CompilerParams(needs_layout_passes=False),
        out_type=jax.ShapeDtypeStruct((_N,), jnp.float32),
        scratch_types=[
            pltpu.VMEM((_GPAD,), jnp.float32),
            pltpu.VMEM((_GPAD,), jnp.float32),
            pltpu.VMEM((_GPAD,), jnp.int32),
            pltpu.VMEM((_CHUNK,), jnp.float32),
            pltpu.VMEM((_CHUNK,), jnp.float32),
            pltpu.VMEM((_CHUNK,), jnp.int32),
            pltpu.VMEM((_CHUNK,), jnp.int32),
            pltpu.VMEM((_CHUNK,), jnp.float32),
            pltpu.VMEM((_CHUNK,), jnp.float32),
            pltpu.SemaphoreType.DMA,
            pltpu.SemaphoreType.DMA,
            pltpu.SemaphoreType.DMA,
            pltpu.SemaphoreType.DMA,
            pltpu.SemaphoreType.DMA,
        ],
    )
    return run(x, g32, means, sds)


# speedup vs baseline: 1.0618x; 1.0618x over previous
import jax
import jax.numpy as jnp
from jax import lax
from jax.experimental import pallas as pl
from jax.experimental.pallas import tpu as pltpu
from jax.experimental.pallas import tpu_sc as plsc

_N = 4_194_304
_G = 1000
_GPAD = 1024
_NC = 2
_NS = 16
_NW = _NC * _NS
_PER_TILE = _N // _NW
_CHUNK = 8192
_STEPS = _PER_TILE // _CHUNK
_NBUF = 3
_L = 16


def _body(x_hbm, g_hbm, means_hbm, sds_hbm, out_hbm,
          tbl_m, tbl_s, tbl_p,
          x_v0, x_v1, x_v2, g_v0, g_v1, g_v2, o_v0, o_v1, o_v2,
          semld0, semld1, semld2, semst0, semst1, semst2, semtb):
    wid = lax.axis_index("s") * _NC + lax.axis_index("c")
    base = wid * _PER_TILE
    x_v = (x_v0, x_v1, x_v2)
    g_v = (g_v0, g_v1, g_v2)
    o_v = (o_v0, o_v1, o_v2)
    semld = (semld0, semld1, semld2)
    semst = (semst0, semst1, semst2)

    def load(c):
        b = c % _NBUF
        off = base + c * _CHUNK
        cx = pltpu.make_async_copy(
            x_hbm.at[pl.ds(off, _CHUNK)], x_v[b], semld[b])
        cg = pltpu.make_async_copy(
            g_hbm.at[pl.ds(off, _CHUNK)], g_v[b], semld[b])
        return cx, cg

    def store(c):
        b = c % _NBUF
        off = base + c * _CHUNK
        return pltpu.make_async_copy(
            o_v[b], out_hbm.at[pl.ds(off, _CHUNK)], semst[b])

    for c in range(_NBUF - 1):
        cx, cg = load(c)
        cx.start()
        cg.start()

    cm = pltpu.make_async_copy(means_hbm, tbl_m.at[pl.ds(0, _G)], semtb)
    cs = pltpu.make_async_copy(sds_hbm, tbl_s.at[pl.ds(0, _G)], semtb)
    cm.start()
    cs.start()
    cm.wait()
    cs.wait()

    def init_tbl(i, carry):
        m = tbl_m[pl.ds(i * _L, _L)]
        s = tbl_s[pl.ds(i * _L, _L)]
        a = 1.0 / s
        b = -(m * a)
        ai = plsc.bitcast(a, jnp.int32) + jnp.int32(0x8000)
        bi = plsc.bitcast(b, jnp.int32) + jnp.int32(0x8000)
        packed = (ai & jnp.int32(-65536)) | (
            lax.shift_right_logical(bi, jnp.int32(16)))
        tbl_p[pl.ds(i * _L, _L)] = packed
        return carry

    lax.fori_loop(0, (_G + _L - 1) // _L, init_tbl, 0)

    for c in range(_STEPS):
        b = c % _NBUF
        if c + _NBUF - 1 < _STEPS:
            nx, ng = load(c + _NBUF - 1)
            nx.start()
            ng.start()
        lx, lg = load(c)
        lx.wait()
        lg.wait()
        if c >= _NBUF:
            store(c - _NBUF).wait()
        gr = g_v[b]
        xr = x_v[b]
        orf = o_v[b]

        @plsc.parallel_loop(0, _CHUNK, step=_L, unroll=4)
        def _inner(e):
            idx = gr[pl.ds(e, _L)] - 1
            p = plsc.load_gather(tbl_p, [idx])
            a = plsc.bitcast(p & jnp.int32(-65536), jnp.float32)
            bb = plsc.bitcast(lax.shift_left(p, jnp.int32(16)), jnp.float32)
            orf[pl.ds(e, _L)] = xr[pl.ds(e, _L)] * a + bb

        store(c).start()

    for c in range(_STEPS - _NBUF, _STEPS):
        store(c).wait()


def kernel(x, group, means, sds):
    g32 = group.astype(jnp.int32)
    mesh = plsc.VectorSubcoreMesh(core_axis_name="c", subcore_axis_name="s")
    run = pl.kernel(
        _body,
        mesh=mesh,
        compiler_params=pltpu.CompilerParams(needs_layout_passes=False),
        out_type=jax.ShapeDtypeStruct((_N,), jnp.float32),
        scratch_types=[
            pltpu.VMEM((_GPAD,), jnp.float32),
            pltpu.VMEM((_GPAD,), jnp.float32),
            pltpu.VMEM((_GPAD,), jnp.int32),
            pltpu.VMEM((_CHUNK,), jnp.float32),
            pltpu.VMEM((_CHUNK,), jnp.float32),
            pltpu.VMEM((_CHUNK,), jnp.float32),
            pltpu.VMEM((_CHUNK,), jnp.int32),
            pltpu.VMEM((_CHUNK,), jnp.int32),
            pltpu.VMEM((_CHUNK,), jnp.int32),
            pltpu.VMEM((_CHUNK,), jnp.float32),
            pltpu.VMEM((_CHUNK,), jnp.float32),
            pltpu.VMEM((_CHUNK,), jnp.float32),
            pltpu.SemaphoreType.DMA,
            pltpu.SemaphoreType.DMA,
            pltpu.SemaphoreType.DMA,
            pltpu.SemaphoreType.DMA,
            pltpu.SemaphoreType.DMA,
            pltpu.SemaphoreType.DMA,
            pltpu.SemaphoreType.DMA,
        ],
    )
    return run(x, g32, means, sds)
